# baseline (speedup 1.0000x reference)
import functools

import jax
import jax.numpy as jnp
from jax import lax
from jax.experimental import pallas as pl
from jax.experimental.pallas import tpu as pltpu
from jax.experimental.pallas import tpu_sc as plsc

N = 10000
E = 320000
D = 128
G = 16

NP = 10240
EP = 322560
NC = 2
NS = 16
NW = NC * NS
EW = EP // NW
K = 120
NB = EW // K
NBUF = 3
NB0 = 156
NB1 = 12
CC = 1680
NSTRIPE = NP // NS
SROWS = NP // NS


def _sc_degree_body(col_hbm, out_hbm, ccv, hist, redbuf, hist_sh):
    c = lax.axis_index("c")
    s = lax.axis_index("s")
    wid = s * NC + c
    base = wid * EW
    ones = jnp.ones((16,), jnp.float32)
    zeros = jnp.zeros((16,), jnp.float32)

    @pl.loop(0, NP // 16)
    def _(v):
        hist[pl.ds(v * 16, 16)] = zeros

    @pl.loop(0, EW // CC)
    def _(ch):
        pltpu.sync_copy(col_hbm.at[pl.ds(base + ch * CC, CC)], ccv)

        @pl.loop(0, CC // 16)
        def _(j):
            idx = ccv[pl.ds(j * 16, 16)]
            plsc.addupdate_scatter(hist, [idx], ones)

    pltpu.sync_copy(hist, hist_sh.at[s])
    plsc.subcore_barrier()
    pltpu.sync_copy(hist_sh.at[:, pl.ds(s * NSTRIPE, NSTRIPE)], redbuf)

    @pl.loop(0, NSTRIPE // 16)
    def _(v):
        tot = redbuf[0, pl.ds(v * 16, 16)]
        for r in range(1, NS):
            tot = tot + redbuf[r, pl.ds(v * 16, 16)]
        hist[pl.ds(v * 16, 16)] = tot

    pltpu.sync_copy(hist.at[pl.ds(0, NSTRIPE)],
                    out_hbm.at[c, pl.ds(s * NSTRIPE, NSTRIPE)])


def _sc_edge_scatter_body(hs_hbm, rc_hbm, zrows_hbm, out_hbm,
                          rcv, rb, gsem, ssem, acc):
    c = lax.axis_index("c")
    s = lax.axis_index("s")
    nb = jnp.where(c == 0, NB0, NB1)
    bbase = jnp.where(c == 0, s * NB0, NS * NB0 + s * NB1)
    pltpu.sync_copy(zrows_hbm, acc.at[pl.ds(s * SROWS, SROWS)])
    plsc.subcore_barrier()

    def start_gather(i, b):
        pltpu.sync_copy(rc_hbm.at[pl.ds(bbase + i, 1)], rcv[b])
        pltpu.async_copy(hs_hbm.at[rcv[b].at[0, 0]], rb[b], gsem[b])

    def fire_scatter(b):
        pltpu.make_async_copy(hs_hbm.at[rcv[b].at[0, 0]], rb[b],
                              gsem[b]).wait()
        pltpu.make_async_copy(rb[b], acc.at[rcv[b].at[0, 1]],
                              ssem[b]).start(add=True)

    def drain_scatter(b):
        pltpu.make_async_copy(rb[b], acc.at[rcv[b].at[0, 1]], ssem[b]).wait()

    for b in range(NBUF):
        start_gather(b, b)

    @pl.loop(0, nb - NBUF, step=NBUF)
    def _(i):
        fire_scatter(0)
        fire_scatter(1)
        for b in range(NBUF - 2):
            drain_scatter(b)
            start_gather(i + NBUF + b, b)
            fire_scatter(b + 2)
        for b in range(NBUF - 2, NBUF):
            drain_scatter(b)
            start_gather(i + NBUF + b, b)

    for b in range(NBUF):
        fire_scatter(b)
    for b in range(NBUF):
        drain_scatter(b)

    plsc.subcore_barrier()
    pltpu.sync_copy(acc.at[pl.ds(s * SROWS, SROWS)],
                    out_hbm.at[c, pl.ds(s * SROWS, SROWS)])


@functools.cache
def _get_sc_kernels():
    mesh = plsc.VectorSubcoreMesh(core_axis_name="c", subcore_axis_name="s",
                                  num_cores=NC, num_subcores=NS)
    sc_degree = pl.kernel(
        _sc_degree_body,
        out_type=jax.ShapeDtypeStruct((NC, NP), jnp.float32),
        mesh=mesh,
        compiler_params=pltpu.CompilerParams(needs_layout_passes=False),
        scratch_types=[
            pltpu.VMEM((CC,), jnp.int32),
            pltpu.VMEM((NP,), jnp.float32),
            pltpu.VMEM((NS, NSTRIPE), jnp.float32),
            pltpu.VMEM_SHARED((NS, NP), jnp.float32),
        ],
    )
    sc_edge_scatter = pl.kernel(
        _sc_edge_scatter_body,
        out_type=jax.ShapeDtypeStruct((NC, NP, D), jnp.float32),
        mesh=mesh,
        scratch_types=[
            [pltpu.VMEM((1, 2, K), jnp.int32) for _ in range(NBUF)],
            [pltpu.VMEM((K, D), jnp.float32) for _ in range(NBUF)],
            [pltpu.SemaphoreType.DMA for _ in range(NBUF)],
            [pltpu.SemaphoreType.DMA for _ in range(NBUF)],
            pltpu.VMEM_SHARED((NP, D), jnp.float32),
        ],
    )
    return sc_degree, sc_edge_scatter


BN = 512
NBLK = NP // BN


def _tc_first_body(degp_ref, x_ref, w_ref, hs_ref, dinv_ref):
    deg = degp_ref[0] + degp_ref[1] + 1.0
    dinv1 = lax.rsqrt(deg)
    dinv = jnp.broadcast_to(dinv1, (BN, D))
    h = jnp.dot(x_ref[...], w_ref[...], preferred_element_type=jnp.float32)
    hs_ref[...] = h * dinv
    dinv_ref[...] = dinv


def _tc_mid_body(accp_ref, hs_ref, dinv_ref, b_ref, w_ref, hsout_ref):
    dinv = dinv_ref[...]
    acc = accp_ref[0] + accp_ref[1]
    x = jnp.maximum(dinv * (acc + hs_ref[...]) + b_ref[...], 0.0)
    hsout_ref[...] = jnp.dot(
        x, w_ref[...], preferred_element_type=jnp.float32) * dinv


def _tc_head_body(accp_ref, hs_ref, dinv_ref, b3_ref, wf1_ref, bf1_ref,
                  wf2_ref, bf2_ref, batch_ref, s_ref, cnt_ref):
    r = pl.program_id(0)

    @pl.when(r == 0)
    def _():
        s_ref[...] = jnp.zeros((G, D), jnp.float32)
        cnt_ref[...] = jnp.zeros((G, D), jnp.float32)

    dinv = dinv_ref[...]
    acc = accp_ref[0] + accp_ref[1]
    x = jnp.maximum(dinv * (acc + hs_ref[...]) + b3_ref[...], 0.0)
    h = jnp.maximum(
        jnp.dot(x, wf1_ref[...], preferred_element_type=jnp.float32)
        + bf1_ref[...], 0.0)
    z = (jnp.dot(h, wf2_ref[...], preferred_element_type=jnp.float32)
         + bf2_ref[...])
    gid = lax.broadcasted_iota(jnp.int32, (G, D), 0)
    for j in range(BN // D):
        bj = jnp.broadcast_to(batch_ref[0, pl.ds(j, 1), :], (G, D))
        oh = (gid == bj).astype(jnp.float32)
        s_ref[...] += jnp.dot(oh, z[j * D:(j + 1) * D, :],
                              preferred_element_type=jnp.float32)
        cnt_ref[...] += jnp.broadcast_to(
            jnp.sum(oh, axis=1, keepdims=True), (G, D))


def _tc_softmax_body(s_ref, cnt_ref, out_ref):
    m = s_ref[...] / jnp.maximum(cnt_ref[...], 1.0)
    d = m[:, 1:2] - m[:, 0:1]
    p0 = jnp.broadcast_to(1.0 / (1.0 + jnp.exp(d)), (G, D))
    p1 = jnp.broadcast_to(1.0 / (1.0 + jnp.exp(-d)), (G, D))
    col = lax.broadcasted_iota(jnp.int32, (G, D), 1)
    out_ref[...] = jnp.where(col == 0, p0, jnp.where(col == 1, p1, 0.0))


def _node_spec(width=D):
    return pl.BlockSpec((BN, width), lambda r: (r, 0))


def _part_spec(width=D):
    return pl.BlockSpec((NC, BN, width), lambda r: (0, r, 0))


def _full_spec(shape):
    return pl.BlockSpec(shape, lambda r: tuple(0 for _ in shape))


_tc_first = pl.pallas_call(
    _tc_first_body,
    grid=(NBLK,),
    in_specs=[_part_spec(1), _node_spec(), _full_spec((D, D))],
    out_specs=[_node_spec(), _node_spec()],
    out_shape=[jax.ShapeDtypeStruct((NP, D), jnp.float32),
               jax.ShapeDtypeStruct((NP, D), jnp.float32)],
)

_tc_mid = pl.pallas_call(
    _tc_mid_body,
    grid=(NBLK,),
    in_specs=[_part_spec(), _node_spec(), _node_spec(),
              _full_spec((1, D)), _full_spec((D, D))],
    out_specs=_node_spec(),
    out_shape=jax.ShapeDtypeStruct((NP, D), jnp.float32),
)

_tc_head = pl.pallas_call(
    _tc_head_body,
    grid=(NBLK,),
    in_specs=[_part_spec(), _node_spec(), _node_spec(),
              _full_spec((1, D)), _full_spec((D, D)), _full_spec((1, D)),
              _full_spec((D, D)), _full_spec((1, D)),
              pl.BlockSpec((1, BN // D, D), lambda r: (r, 0, 0))],
    out_specs=[_full_spec((G, D)), _full_spec((G, D))],
    out_shape=[jax.ShapeDtypeStruct((G, D), jnp.float32),
               jax.ShapeDtypeStruct((G, D), jnp.float32)],
)

_tc_softmax = pl.pallas_call(
    _tc_softmax_body,
    in_specs=[pl.BlockSpec((G, D), lambda: (0, 0)),
              pl.BlockSpec((G, D), lambda: (0, 0))],
    out_specs=pl.BlockSpec((G, D), lambda: (0, 0)),
    out_shape=jax.ShapeDtypeStruct((G, D), jnp.float32),
)


def kernel(x, edge_index, batch, W1, b1, W2, b2, W3, b3, Wf1, bf1, Wf2, bf2):
    f32 = jnp.float32
    pad_e = jnp.full((EP - E,), N, jnp.int32)
    row = jnp.concatenate([edge_index[0].astype(jnp.int32), pad_e])
    col = jnp.concatenate([edge_index[1].astype(jnp.int32), pad_e])
    xp = jnp.concatenate([x, jnp.zeros((NP - N, D), f32)], axis=0)
    batch_p = jnp.concatenate(
        [batch.astype(jnp.int32), jnp.full((NP - N,), G, jnp.int32)]
    ).reshape(NBLK, BN // D, D)

    rc = jnp.stack([row.reshape(EP // K, K), col.reshape(EP // K, K)], axis=1)
    zrows = jnp.zeros((SROWS, D), f32)
    b1r, b2r, b3r, bf1r = (v.reshape(1, D) for v in (b1, b2, b3, bf1))
    wf2p = jnp.zeros((D, D), f32).at[:, :2].set(Wf2)
    bf2p = jnp.zeros((1, D), f32).at[0, :2].set(bf2)

    sc_degree, sc_edge_scatter = _get_sc_kernels()
    degp = sc_degree(col).reshape(NC, NP, 1)
    hs1, dinv = _tc_first(degp, xp, W1)
    acc1 = sc_edge_scatter(hs1, rc, zrows)
    hs2 = _tc_mid(acc1, hs1, dinv, b1r, W2)
    acc2 = sc_edge_scatter(hs2, rc, zrows)
    hs3 = _tc_mid(acc2, hs2, dinv, b2r, W3)
    acc3 = sc_edge_scatter(hs3, rc, zrows)
    s, cnt = _tc_head(acc3, hs3, dinv, b3r, Wf1, bf1r, wf2p, bf2p, batch_p)
    out = _tc_softmax(s, cnt)
    return out[:, :2]


# speedup vs baseline: 1.0177x; 1.0177x over previous
import functools

import jax
import jax.numpy as jnp
from jax import lax
from jax.experimental import pallas as pl
from jax.experimental.pallas import tpu as pltpu
from jax.experimental.pallas import tpu_sc as plsc

N = 10000
E = 320000
D = 128
G = 16

NP = 10240
EP = 322560
NC = 2
NS = 16
NW = NC * NS
EW = EP // NW
K = 120
NB = EW // K
NBUF = 3
NB0 = 147
NB1 = 21
CC = 1680
NSTRIPE = NP // NS
SROWS = NP // NS


def _sc_degree_body(col_hbm, out_hbm, ccv, hist, redbuf, hist_sh):
    c = lax.axis_index("c")
    s = lax.axis_index("s")
    wid = s * NC + c
    base = wid * EW
    ones = jnp.ones((16,), jnp.float32)
    zeros = jnp.zeros((16,), jnp.float32)

    @pl.loop(0, NP // 16)
    def _(v):
        hist[pl.ds(v * 16, 16)] = zeros

    @pl.loop(0, EW // CC)
    def _(ch):
        pltpu.sync_copy(col_hbm.at[pl.ds(base + ch * CC, CC)], ccv)

        @pl.loop(0, CC // 16)
        def _(j):
            idx = ccv[pl.ds(j * 16, 16)]
            plsc.addupdate_scatter(hist, [idx], ones)

    pltpu.sync_copy(hist, hist_sh.at[s])
    plsc.subcore_barrier()
    pltpu.sync_copy(hist_sh.at[:, pl.ds(s * NSTRIPE, NSTRIPE)], redbuf)

    @pl.loop(0, NSTRIPE // 16)
    def _(v):
        tot = redbuf[0, pl.ds(v * 16, 16)]
        for r in range(1, NS):
            tot = tot + redbuf[r, pl.ds(v * 16, 16)]
        hist[pl.ds(v * 16, 16)] = tot

    pltpu.sync_copy(hist.at[pl.ds(0, NSTRIPE)],
                    out_hbm.at[c, pl.ds(s * NSTRIPE, NSTRIPE)])


def _sc_edge_scatter_body(hs_hbm, rc_hbm, zrows_hbm, out_hbm,
                          rcv, rb, gsem, ssem, acc):
    c = lax.axis_index("c")
    s = lax.axis_index("s")
    nb = jnp.where(c == 0, NB0, NB1)
    bbase = jnp.where(c == 0, s * NB0, NS * NB0 + s * NB1)
    pltpu.sync_copy(zrows_hbm, acc.at[pl.ds(s * SROWS, SROWS)])
    plsc.subcore_barrier()

    def start_gather(i, b):
        pltpu.sync_copy(rc_hbm.at[pl.ds(bbase + i, 1)], rcv[b])
        pltpu.async_copy(hs_hbm.at[rcv[b].at[0, 0]], rb[b], gsem[b])

    def fire_scatter(b):
        pltpu.make_async_copy(hs_hbm.at[rcv[b].at[0, 0]], rb[b],
                              gsem[b]).wait()
        pltpu.make_async_copy(rb[b], acc.at[rcv[b].at[0, 1]],
                              ssem[b]).start(add=True)

    def drain_scatter(b):
        pltpu.make_async_copy(rb[b], acc.at[rcv[b].at[0, 1]], ssem[b]).wait()

    for b in range(NBUF):
        start_gather(b, b)

    @pl.loop(0, nb - NBUF, step=NBUF)
    def _(i):
        fire_scatter(0)
        fire_scatter(1)
        for b in range(NBUF - 2):
            drain_scatter(b)
            start_gather(i + NBUF + b, b)
            fire_scatter(b + 2)
        for b in range(NBUF - 2, NBUF):
            drain_scatter(b)
            start_gather(i + NBUF + b, b)

    for b in range(NBUF):
        fire_scatter(b)
    for b in range(NBUF):
        drain_scatter(b)

    plsc.subcore_barrier()
    pltpu.sync_copy(acc.at[pl.ds(s * SROWS, SROWS)],
                    out_hbm.at[c, pl.ds(s * SROWS, SROWS)])


@functools.cache
def _get_sc_kernels():
    mesh = plsc.VectorSubcoreMesh(core_axis_name="c", subcore_axis_name="s",
                                  num_cores=NC, num_subcores=NS)
    sc_degree = pl.kernel(
        _sc_degree_body,
        out_type=jax.ShapeDtypeStruct((NC, NP), jnp.float32),
        mesh=mesh,
        compiler_params=pltpu.CompilerParams(needs_layout_passes=False),
        scratch_types=[
            pltpu.VMEM((CC,), jnp.int32),
            pltpu.VMEM((NP,), jnp.float32),
            pltpu.VMEM((NS, NSTRIPE), jnp.float32),
            pltpu.VMEM_SHARED((NS, NP), jnp.float32),
        ],
    )
    sc_edge_scatter = pl.kernel(
        _sc_edge_scatter_body,
        out_type=jax.ShapeDtypeStruct((NC, NP, D), jnp.float32),
        mesh=mesh,
        scratch_types=[
            [pltpu.VMEM((1, 2, K), jnp.int32) for _ in range(NBUF)],
            [pltpu.VMEM((K, D), jnp.float32) for _ in range(NBUF)],
            [pltpu.SemaphoreType.DMA for _ in range(NBUF)],
            [pltpu.SemaphoreType.DMA for _ in range(NBUF)],
            pltpu.VMEM_SHARED((NP, D), jnp.float32),
        ],
    )
    return sc_degree, sc_edge_scatter


BN = 512
NBLK = NP // BN


def _tc_first_body(degp_ref, x_ref, w_ref, hs_ref, dinv_ref):
    deg = degp_ref[0] + degp_ref[1] + 1.0
    dinv1 = lax.rsqrt(deg)
    dinv = jnp.broadcast_to(dinv1, (BN, D))
    h = jnp.dot(x_ref[...], w_ref[...], preferred_element_type=jnp.float32)
    hs_ref[...] = h * dinv
    dinv_ref[...] = dinv


def _tc_mid_body(accp_ref, hs_ref, dinv_ref, b_ref, w_ref, hsout_ref):
    dinv = dinv_ref[...]
    acc = accp_ref[0] + accp_ref[1]
    x = jnp.maximum(dinv * (acc + hs_ref[...]) + b_ref[...], 0.0)
    hsout_ref[...] = jnp.dot(
        x, w_ref[...], preferred_element_type=jnp.float32) * dinv


def _tc_head_body(accp_ref, hs_ref, dinv_ref, b3_ref, wf1_ref, bf1_ref,
                  wf2_ref, bf2_ref, batch_ref, s_ref, cnt_ref):
    r = pl.program_id(0)

    @pl.when(r == 0)
    def _():
        s_ref[...] = jnp.zeros((G, D), jnp.float32)
        cnt_ref[...] = jnp.zeros((G, D), jnp.float32)

    dinv = dinv_ref[...]
    acc = accp_ref[0] + accp_ref[1]
    x = jnp.maximum(dinv * (acc + hs_ref[...]) + b3_ref[...], 0.0)
    h = jnp.maximum(
        jnp.dot(x, wf1_ref[...], preferred_element_type=jnp.float32)
        + bf1_ref[...], 0.0)
    z = (jnp.dot(h, wf2_ref[...], preferred_element_type=jnp.float32)
         + bf2_ref[...])
    gid = lax.broadcasted_iota(jnp.int32, (G, D), 0)
    for j in range(BN // D):
        bj = jnp.broadcast_to(batch_ref[0, pl.ds(j, 1), :], (G, D))
        oh = (gid == bj).astype(jnp.float32)
        s_ref[...] += jnp.dot(oh, z[j * D:(j + 1) * D, :],
                              preferred_element_type=jnp.float32)
        cnt_ref[...] += jnp.broadcast_to(
            jnp.sum(oh, axis=1, keepdims=True), (G, D))


def _tc_softmax_body(s_ref, cnt_ref, out_ref):
    m = s_ref[...] / jnp.maximum(cnt_ref[...], 1.0)
    d = m[:, 1:2] - m[:, 0:1]
    p0 = jnp.broadcast_to(1.0 / (1.0 + jnp.exp(d)), (G, D))
    p1 = jnp.broadcast_to(1.0 / (1.0 + jnp.exp(-d)), (G, D))
    col = lax.broadcasted_iota(jnp.int32, (G, D), 1)
    out_ref[...] = jnp.where(col == 0, p0, jnp.where(col == 1, p1, 0.0))


def _node_spec(width=D):
    return pl.BlockSpec((BN, width), lambda r: (r, 0))


def _part_spec(width=D):
    return pl.BlockSpec((NC, BN, width), lambda r: (0, r, 0))


def _full_spec(shape):
    return pl.BlockSpec(shape, lambda r: tuple(0 for _ in shape))


_tc_first = pl.pallas_call(
    _tc_first_body,
    grid=(NBLK,),
    in_specs=[_part_spec(1), _node_spec(), _full_spec((D, D))],
    out_specs=[_node_spec(), _node_spec()],
    out_shape=[jax.ShapeDtypeStruct((NP, D), jnp.float32),
               jax.ShapeDtypeStruct((NP, D), jnp.float32)],
)

_tc_mid = pl.pallas_call(
    _tc_mid_body,
    grid=(NBLK,),
    in_specs=[_part_spec(), _node_spec(), _node_spec(),
              _full_spec((1, D)), _full_spec((D, D))],
    out_specs=_node_spec(),
    out_shape=jax.ShapeDtypeStruct((NP, D), jnp.float32),
)

_tc_head = pl.pallas_call(
    _tc_head_body,
    grid=(NBLK,),
    in_specs=[_part_spec(), _node_spec(), _node_spec(),
              _full_spec((1, D)), _full_spec((D, D)), _full_spec((1, D)),
              _full_spec((D, D)), _full_spec((1, D)),
              pl.BlockSpec((1, BN // D, D), lambda r: (r, 0, 0))],
    out_specs=[_full_spec((G, D)), _full_spec((G, D))],
    out_shape=[jax.ShapeDtypeStruct((G, D), jnp.float32),
               jax.ShapeDtypeStruct((G, D), jnp.float32)],
)

_tc_softmax = pl.pallas_call(
    _tc_softmax_body,
    in_specs=[pl.BlockSpec((G, D), lambda: (0, 0)),
              pl.BlockSpec((G, D), lambda: (0, 0))],
    out_specs=pl.BlockSpec((G, D), lambda: (0, 0)),
    out_shape=jax.ShapeDtypeStruct((G, D), jnp.float32),
)


def kernel(x, edge_index, batch, W1, b1, W2, b2, W3, b3, Wf1, bf1, Wf2, bf2):
    f32 = jnp.float32
    pad_e = jnp.full((EP - E,), N, jnp.int32)
    row = jnp.concatenate([edge_index[0].astype(jnp.int32), pad_e])
    col = jnp.concatenate([edge_index[1].astype(jnp.int32), pad_e])
    xp = jnp.concatenate([x, jnp.zeros((NP - N, D), f32)], axis=0)
    batch_p = jnp.concatenate(
        [batch.astype(jnp.int32), jnp.full((NP - N,), G, jnp.int32)]
    ).reshape(NBLK, BN // D, D)

    rc = jnp.stack([row.reshape(EP // K, K), col.reshape(EP // K, K)], axis=1)
    zrows = jnp.zeros((SROWS, D), f32)
    b1r, b2r, b3r, bf1r = (v.reshape(1, D) for v in (b1, b2, b3, bf1))
    wf2p = jnp.zeros((D, D), f32).at[:, :2].set(Wf2)
    bf2p = jnp.zeros((1, D), f32).at[0, :2].set(bf2)

    sc_degree, sc_edge_scatter = _get_sc_kernels()
    degp = sc_degree(col).reshape(NC, NP, 1)
    hs1, dinv = _tc_first(degp, xp, W1)
    acc1 = sc_edge_scatter(hs1, rc, zrows)
    hs2 = _tc_mid(acc1, hs1, dinv, b1r, W2)
    acc2 = sc_edge_scatter(hs2, rc, zrows)
    hs3 = _tc_mid(acc2, hs2, dinv, b2r, W3)
    acc3 = sc_edge_scatter(hs3, rc, zrows)
    s, cnt = _tc_head(acc3, hs3, dinv, b3r, Wf1, bf1r, wf2p, bf2p, batch_p)
    out = _tc_softmax(s, cnt)
    return out[:, :2]


# speedup vs baseline: 1.0487x; 1.0304x over previous
import functools

import jax
import jax.numpy as jnp
from jax import lax
from jax.experimental import pallas as pl
from jax.experimental.pallas import tpu as pltpu
from jax.experimental.pallas import tpu_sc as plsc

N = 10000
E = 320000
D = 128
G = 16

NP = 10240
EP = 322560
NC = 2
NS = 16
NW = NC * NS
EW = EP // NW
K = 120
NB = EW // K
NBUF = 3
NB0 = 147
NB1 = 21
CC = 1680
NSTRIPE = NP // NS
SROWS = NP // NS


def _sc_degree_body(col_hbm, out_hbm, ccv, hist, redbuf, hist_sh):
    c = lax.axis_index("c")
    s = lax.axis_index("s")
    wid = s * NC + c
    base = wid * EW
    ones = jnp.ones((16,), jnp.float32)
    zeros = jnp.zeros((16,), jnp.float32)

    @pl.loop(0, NP // 16)
    def _(v):
        hist[pl.ds(v * 16, 16)] = zeros

    @pl.loop(0, EW // CC)
    def _(ch):
        pltpu.sync_copy(col_hbm.at[pl.ds(base + ch * CC, CC)], ccv)

        @pl.loop(0, CC // 16)
        def _(j):
            idx = ccv[pl.ds(j * 16, 16)]
            plsc.addupdate_scatter(hist, [idx], ones)

    pltpu.sync_copy(hist, hist_sh.at[s])
    plsc.subcore_barrier()
    pltpu.sync_copy(hist_sh.at[:, pl.ds(s * NSTRIPE, NSTRIPE)], redbuf)

    @pl.loop(0, NSTRIPE // 16)
    def _(v):
        tot = redbuf[0, pl.ds(v * 16, 16)]
        for r in range(1, NS):
            tot = tot + redbuf[r, pl.ds(v * 16, 16)]
        hist[pl.ds(v * 16, 16)] = tot

    pltpu.sync_copy(hist.at[pl.ds(0, NSTRIPE)],
                    out_hbm.at[c, pl.ds(s * NSTRIPE, NSTRIPE)])


def _sc_edge_scatter_body(hs_hbm, rc_hbm, zrows_hbm, out_hbm,
                          rcv, rb, gsem, ssem, acc):
    c = lax.axis_index("c")
    s = lax.axis_index("s")
    nb = jnp.where(c == 0, NB0, NB1)
    bbase = jnp.where(c == 0, s * NB0, NS * NB0 + s * NB1)
    pltpu.sync_copy(zrows_hbm, acc.at[pl.ds(s * SROWS, SROWS)])
    plsc.subcore_barrier()

    def start_gather(i, b):
        pltpu.sync_copy(rc_hbm.at[pl.ds(bbase + i, 1)], rcv[b])
        pltpu.async_copy(hs_hbm.at[rcv[b].at[0, 0]], rb[b], gsem[b])

    def fire_scatter(b):
        pltpu.make_async_copy(hs_hbm.at[rcv[b].at[0, 0]], rb[b],
                              gsem[b]).wait()
        pltpu.make_async_copy(rb[b], acc.at[rcv[b].at[0, 1]],
                              ssem[b]).start(add=True)

    def drain_scatter(b):
        pltpu.make_async_copy(rb[b], acc.at[rcv[b].at[0, 1]], ssem[b]).wait()

    for b in range(NBUF):
        start_gather(b, b)

    @pl.loop(0, nb - NBUF, step=NBUF)
    def _(i):
        fire_scatter(0)
        fire_scatter(1)
        for b in range(NBUF - 2):
            drain_scatter(b)
            start_gather(i + NBUF + b, b)
            fire_scatter(b + 2)
        for b in range(NBUF - 2, NBUF):
            drain_scatter(b)
            start_gather(i + NBUF + b, b)

    for b in range(NBUF):
        fire_scatter(b)
    for b in range(NBUF):
        drain_scatter(b)

    plsc.subcore_barrier()
    pltpu.sync_copy(acc.at[pl.ds(s * SROWS, SROWS)],
                    out_hbm.at[c, pl.ds(s * SROWS, SROWS)])


@functools.cache
def _get_sc_kernels():
    mesh = plsc.VectorSubcoreMesh(core_axis_name="c", subcore_axis_name="s",
                                  num_cores=NC, num_subcores=NS)
    sc_degree = pl.kernel(
        _sc_degree_body,
        out_type=jax.ShapeDtypeStruct((NC, NP), jnp.float32),
        mesh=mesh,
        compiler_params=pltpu.CompilerParams(needs_layout_passes=False),
        scratch_types=[
            pltpu.VMEM((CC,), jnp.int32),
            pltpu.VMEM((NP,), jnp.float32),
            pltpu.VMEM((NS, NSTRIPE), jnp.float32),
            pltpu.VMEM_SHARED((NS, NP), jnp.float32),
        ],
    )
    sc_edge_scatter = pl.kernel(
        _sc_edge_scatter_body,
        out_type=jax.ShapeDtypeStruct((NC, NP, D), jnp.float32),
        mesh=mesh,
        scratch_types=[
            [pltpu.VMEM((1, 2, K), jnp.int32) for _ in range(NBUF)],
            [pltpu.VMEM((K, D), jnp.float32) for _ in range(NBUF)],
            [pltpu.SemaphoreType.DMA for _ in range(NBUF)],
            [pltpu.SemaphoreType.DMA for _ in range(NBUF)],
            pltpu.VMEM_SHARED((NP, D), jnp.float32),
        ],
    )
    return sc_degree, sc_edge_scatter


BN = 1024
NBLK = NP // BN


def _dinv_block(degp_ref):
    deg = degp_ref[0] + degp_ref[1] + 1.0
    return jnp.broadcast_to(lax.rsqrt(deg), (BN, D))


def _tc_first_body(degp_ref, x_ref, w_ref, hs_ref):
    h = jnp.dot(x_ref[...], w_ref[...], preferred_element_type=jnp.float32)
    hs_ref[...] = h * _dinv_block(degp_ref)


def _tc_mid_body(degp_ref, accp_ref, hs_ref, b_ref, w_ref, hsout_ref):
    dinv = _dinv_block(degp_ref)
    acc = accp_ref[0] + accp_ref[1]
    x = jnp.maximum(dinv * (acc + hs_ref[...]) + b_ref[...], 0.0)
    hsout_ref[...] = jnp.dot(
        x, w_ref[...], preferred_element_type=jnp.float32) * dinv


def _tc_head_body(degp_ref, accp_ref, hs_ref, b3_ref, wf1_ref, bf1_ref,
                  wf2_ref, bf2_ref, batch_ref, s_ref, cnt_ref):
    r = pl.program_id(0)

    @pl.when(r == 0)
    def _():
        s_ref[...] = jnp.zeros((G, D), jnp.float32)
        cnt_ref[...] = jnp.zeros((G, D), jnp.float32)

    dinv = _dinv_block(degp_ref)
    acc = accp_ref[0] + accp_ref[1]
    x = jnp.maximum(dinv * (acc + hs_ref[...]) + b3_ref[...], 0.0)
    h = jnp.maximum(
        jnp.dot(x, wf1_ref[...], preferred_element_type=jnp.float32)
        + bf1_ref[...], 0.0)
    z = (jnp.dot(h, wf2_ref[...], preferred_element_type=jnp.float32)
         + bf2_ref[...])
    gid = lax.broadcasted_iota(jnp.int32, (G, D), 0)
    for j in range(BN // D):
        bj = jnp.broadcast_to(batch_ref[0, pl.ds(j, 1), :], (G, D))
        oh = (gid == bj).astype(jnp.float32)
        s_ref[...] += jnp.dot(oh, z[j * D:(j + 1) * D, :],
                              preferred_element_type=jnp.float32)
        cnt_ref[...] += jnp.broadcast_to(
            jnp.sum(oh, axis=1, keepdims=True), (G, D))


def _tc_softmax_body(s_ref, cnt_ref, out_ref):
    m = s_ref[...] / jnp.maximum(cnt_ref[...], 1.0)
    d = m[:, 1:2] - m[:, 0:1]
    p0 = jnp.broadcast_to(1.0 / (1.0 + jnp.exp(d)), (G, D))
    p1 = jnp.broadcast_to(1.0 / (1.0 + jnp.exp(-d)), (G, D))
    col = lax.broadcasted_iota(jnp.int32, (G, D), 1)
    out_ref[...] = jnp.where(col == 0, p0, jnp.where(col == 1, p1, 0.0))


def _node_spec(width=D):
    return pl.BlockSpec((BN, width), lambda r: (r, 0))


def _part_spec(width=D):
    return pl.BlockSpec((NC, BN, width), lambda r: (0, r, 0))


def _full_spec(shape):
    return pl.BlockSpec(shape, lambda r: tuple(0 for _ in shape))


_tc_first = pl.pallas_call(
    _tc_first_body,
    grid=(NBLK,),
    in_specs=[_part_spec(1), _node_spec(), _full_spec((D, D))],
    out_specs=_node_spec(),
    out_shape=jax.ShapeDtypeStruct((NP, D), jnp.float32),
)

_tc_mid = pl.pallas_call(
    _tc_mid_body,
    grid=(NBLK,),
    in_specs=[_part_spec(1), _part_spec(), _node_spec(),
              _full_spec((1, D)), _full_spec((D, D))],
    out_specs=_node_spec(),
    out_shape=jax.ShapeDtypeStruct((NP, D), jnp.float32),
)

_tc_head = pl.pallas_call(
    _tc_head_body,
    grid=(NBLK,),
    in_specs=[_part_spec(1), _part_spec(), _node_spec(),
              _full_spec((1, D)), _full_spec((D, D)), _full_spec((1, D)),
              _full_spec((D, D)), _full_spec((1, D)),
              pl.BlockSpec((1, BN // D, D), lambda r: (r, 0, 0))],
    out_specs=[_full_spec((G, D)), _full_spec((G, D))],
    out_shape=[jax.ShapeDtypeStruct((G, D), jnp.float32),
               jax.ShapeDtypeStruct((G, D), jnp.float32)],
)

_tc_softmax = pl.pallas_call(
    _tc_softmax_body,
    in_specs=[pl.BlockSpec((G, D), lambda: (0, 0)),
              pl.BlockSpec((G, D), lambda: (0, 0))],
    out_specs=pl.BlockSpec((G, D), lambda: (0, 0)),
    out_shape=jax.ShapeDtypeStruct((G, D), jnp.float32),
)


def kernel(x, edge_index, batch, W1, b1, W2, b2, W3, b3, Wf1, bf1, Wf2, bf2):
    f32 = jnp.float32
    pad_e = jnp.full((EP - E,), N, jnp.int32)
    row = jnp.concatenate([edge_index[0].astype(jnp.int32), pad_e])
    col = jnp.concatenate([edge_index[1].astype(jnp.int32), pad_e])
    xp = jnp.concatenate([x, jnp.zeros((NP - N, D), f32)], axis=0)
    batch_p = jnp.concatenate(
        [batch.astype(jnp.int32), jnp.full((NP - N,), G, jnp.int32)]
    ).reshape(NBLK, BN // D, D)

    rc = jnp.stack([row.reshape(EP // K, K), col.reshape(EP // K, K)], axis=1)
    zrows = jnp.zeros((SROWS, D), f32)
    b1r, b2r, b3r, bf1r = (v.reshape(1, D) for v in (b1, b2, b3, bf1))
    wf2p = jnp.zeros((D, D), f32).at[:, :2].set(Wf2)
    bf2p = jnp.zeros((1, D), f32).at[0, :2].set(bf2)

    sc_degree, sc_edge_scatter = _get_sc_kernels()
    degp = sc_degree(col).reshape(NC, NP, 1)
    hs1 = _tc_first(degp, xp, W1)
    acc1 = sc_edge_scatter(hs1, rc, zrows)
    hs2 = _tc_mid(degp, acc1, hs1, b1r, W2)
    acc2 = sc_edge_scatter(hs2, rc, zrows)
    hs3 = _tc_mid(degp, acc2, hs2, b2r, W3)
    acc3 = sc_edge_scatter(hs3, rc, zrows)
    s, cnt = _tc_head(degp, acc3, hs3, b3r, Wf1, bf1r, wf2p, bf2p, batch_p)
    out = _tc_softmax(s, cnt)
    return out[:, :2]
